# baseline (speedup 1.0000x reference)
import functools

import jax
import jax.numpy as jnp
from jax import lax
from jax.experimental import pallas as pl
from jax.experimental.pallas import tpu as pltpu
from jax.experimental.pallas import tpu_sc as plsc

_B = 16384
_T = 10
_F = 5
_NC = 2
_NS = 16
_NW = _NC * _NS
_BPW = _B // _NW
_L = 16
_CHUNKS = _BPW // _L


def _sc_body(idx_hbm, tab_hbm, out_hbm, idx_v, tab_v, out_v):
    wid = lax.axis_index("s") * _NC + lax.axis_index("c")
    base = wid * _BPW
    pltpu.sync_copy(tab_hbm, tab_v)
    pltpu.sync_copy(idx_hbm.at[pl.ds(base * _T, _BPW * _T)], idx_v)

    lane = lax.iota(jnp.int32, _L)
    lane_t = lane * _T
    lane_f = lane * _F
    tsplat = [jnp.full((_L,), t, jnp.int32) for t in range(_T)]

    def chunk(c, carry):
        rows_t = c * (_L * _T) + lane_t
        rows_f = c * (_L * _F) + lane_f
        acc0 = [None] * _F
        acc1 = [None] * _F
        for t in range(_T):
            idx5 = plsc.load_gather(idx_v, [rows_t + tsplat[t]]) * _F
            for f in range(_F):
                v = plsc.load_gather(tab_v, [idx5 + f])
                if t < 5:
                    acc0[f] = v if acc0[f] is None else acc0[f] * v
                else:
                    acc1[f] = v if acc1[f] is None else acc1[f] * v
        for f in range(_F):
            plsc.store_scatter(out_v, [rows_f + f], (acc0[f] + acc1[f]) * 0.5)
        return carry

    lax.fori_loop(0, _CHUNKS, chunk, 0)
    pltpu.sync_copy(out_v, out_hbm.at[pl.ds(base * _F, _BPW * _F)])


_sc_kernel = functools.partial(
    pl.kernel,
    out_type=jax.ShapeDtypeStruct((_B * _F,), jnp.float32),
    mesh=plsc.VectorSubcoreMesh(core_axis_name="c", subcore_axis_name="s"),
    compiler_params=pltpu.CompilerParams(needs_layout_passes=False),
    scratch_types=[
        pltpu.VMEM((_BPW * _T,), jnp.int32),
        pltpu.VMEM((_F * 20,), jnp.float32),
        pltpu.VMEM((_BPW * _F,), jnp.float32),
    ],
)(_sc_body)


def kernel(inputs, table):
    idx_flat = inputs.reshape(-1).astype(jnp.int32)
    tab_flat = table.reshape(-1)
    return _sc_kernel(idx_flat, tab_flat).reshape(_B, _F)


# speedup vs baseline: 2.2925x; 2.2925x over previous
import functools

import jax
import jax.numpy as jnp
from jax import lax
from jax.experimental import pallas as pl
from jax.experimental.pallas import tpu as pltpu
from jax.experimental.pallas import tpu_sc as plsc

_B = 16384
_T = 10
_F = 5
_V = 20
_NC = 2
_NS = 16
_NW = _NC * _NS
_BPW = _B // _NW
_L = 16
_CHUNKS = _BPW // _L


def _sc_body(idx_hbm, tab_hbm, out_hbm, idx_v, tab_v, out_v):
    wid = lax.axis_index("s") * _NC + lax.axis_index("c")
    base = wid * _BPW
    pltpu.sync_copy(tab_hbm, tab_v)
    pltpu.sync_copy(idx_hbm.at[:, pl.ds(base, _BPW)], idx_v)

    fsplat = [jnp.full((_L,), f, jnp.int32) for f in range(_F)]

    def chunk(c, carry):
        col = c * _L
        acc0 = [None] * _F
        acc1 = [None] * _F
        for t in range(_T):
            idx16 = idx_v[t, pl.ds(col, _L)]
            for f in range(_F):
                v = plsc.load_gather(tab_v, [fsplat[f], idx16])
                if t < 5:
                    acc0[f] = v if acc0[f] is None else acc0[f] * v
                else:
                    acc1[f] = v if acc1[f] is None else acc1[f] * v
        for f in range(_F):
            out_v[f, pl.ds(col, _L)] = (acc0[f] + acc1[f]) * 0.5
        return carry

    lax.fori_loop(0, _CHUNKS, chunk, 0)
    pltpu.sync_copy(out_v, out_hbm.at[:, pl.ds(base, _BPW)])


_sc_kernel = functools.partial(
    pl.kernel,
    out_type=jax.ShapeDtypeStruct((_F, _B), jnp.float32),
    mesh=plsc.VectorSubcoreMesh(core_axis_name="c", subcore_axis_name="s"),
    compiler_params=pltpu.CompilerParams(needs_layout_passes=False),
    scratch_types=[
        pltpu.VMEM((_T, _BPW), jnp.int32),
        pltpu.VMEM((_F, _V), jnp.float32),
        pltpu.VMEM((_F, _BPW), jnp.float32),
    ],
)(_sc_body)


def kernel(inputs, table):
    out_t = _sc_kernel(inputs.T.astype(jnp.int32), table.T)
    return out_t.T


# speedup vs baseline: 2.2935x; 1.0004x over previous
import functools

import jax
import jax.numpy as jnp
from jax import lax
from jax.experimental import pallas as pl
from jax.experimental.pallas import tpu as pltpu
from jax.experimental.pallas import tpu_sc as plsc

_B = 16384
_T = 10
_F = 5
_V = 20
_NC = 2
_NS = 16
_NW = _NC * _NS
_BPW = _B // _NW
_L = 16
_CHUNKS = _BPW // _L


def _sc_body(idx_hbm, tab_hbm, out_hbm, idx_v, tab_v, out_v):
    wid = lax.axis_index("s") * _NC + lax.axis_index("c")
    base = wid * _BPW
    pltpu.sync_copy(tab_hbm, tab_v)
    pltpu.sync_copy(idx_hbm.at[:, pl.ds(base, _BPW)], idx_v)

    fsplat = [jnp.full((_L,), f, jnp.int32) for f in range(_F)]

    @plsc.parallel_loop(0, _CHUNKS, step=1, unroll=2)
    def chunk(c):
        col = c * _L
        acc0 = [None] * _F
        acc1 = [None] * _F
        for t in range(_T):
            idx16 = idx_v[t, pl.ds(col, _L)]
            for f in range(_F):
                v = plsc.load_gather(tab_v, [fsplat[f], idx16])
                if t < 5:
                    acc0[f] = v if acc0[f] is None else acc0[f] * v
                else:
                    acc1[f] = v if acc1[f] is None else acc1[f] * v
        for f in range(_F):
            out_v[f, pl.ds(col, _L)] = (acc0[f] + acc1[f]) * 0.5
    pltpu.sync_copy(out_v, out_hbm.at[:, pl.ds(base, _BPW)])


_sc_kernel = functools.partial(
    pl.kernel,
    out_type=jax.ShapeDtypeStruct((_F, _B), jnp.float32),
    mesh=plsc.VectorSubcoreMesh(core_axis_name="c", subcore_axis_name="s"),
    compiler_params=pltpu.CompilerParams(needs_layout_passes=False),
    scratch_types=[
        pltpu.VMEM((_T, _BPW), jnp.int32),
        pltpu.VMEM((_F, _V), jnp.float32),
        pltpu.VMEM((_F, _BPW), jnp.float32),
    ],
)(_sc_body)


def kernel(inputs, table):
    out_t = _sc_kernel(inputs.T.astype(jnp.int32), table.T)
    return out_t.T


# speedup vs baseline: 2.3106x; 1.0074x over previous
import functools

import jax
import jax.numpy as jnp
from jax import lax
from jax.experimental import pallas as pl
from jax.experimental.pallas import tpu as pltpu
from jax.experimental.pallas import tpu_sc as plsc

_B = 16384
_T = 10
_F = 5
_V = 20
_NC = 2
_NS = 16
_NW = _NC * _NS
_BPW = _B // _NW
_L = 16
_CHUNKS = _BPW // _L


def _sc_body(idx_hbm, tab_hbm, out_hbm, idx_v, tab_v, out_v,
             sem_t, sem_a, sem_b, sem_o):
    wid = lax.axis_index("s") * _NC + lax.axis_index("c")
    base = wid * _BPW
    half = _BPW // 2
    ctab = pltpu.async_copy(tab_hbm, tab_v, sem_t)
    cid_a = pltpu.async_copy(idx_hbm.at[:, pl.ds(base, half)],
                             idx_v.at[:, pl.ds(0, half)], sem_a)
    cid_b = pltpu.async_copy(idx_hbm.at[:, pl.ds(base + half, half)],
                             idx_v.at[:, pl.ds(half, half)], sem_b)

    fsplat = [jnp.full((_L,), f, jnp.int32) for f in range(_F)]

    def make_half(lo, hi):
        @plsc.parallel_loop(lo, hi, step=1, unroll=2)
        def chunk(c):
            col = c * _L
            acc0 = [None] * _F
            acc1 = [None] * _F
            for t in range(_T):
                idx16 = idx_v[t, pl.ds(col, _L)]
                for f in range(_F):
                    v = plsc.load_gather(tab_v, [fsplat[f], idx16])
                    if t < 5:
                        acc0[f] = v if acc0[f] is None else acc0[f] * v
                    else:
                        acc1[f] = v if acc1[f] is None else acc1[f] * v
            for f in range(_F):
                out_v[f, pl.ds(col, _L)] = (acc0[f] + acc1[f]) * 0.5

    ctab.wait()
    cid_a.wait()
    make_half(0, _CHUNKS // 2)
    cout_a = pltpu.async_copy(out_v.at[:, pl.ds(0, half)],
                              out_hbm.at[:, pl.ds(base, half)], sem_o)
    cid_b.wait()
    make_half(_CHUNKS // 2, _CHUNKS)
    cout_a.wait()
    pltpu.sync_copy(out_v.at[:, pl.ds(half, half)],
                    out_hbm.at[:, pl.ds(base + half, half)])


_sc_kernel = functools.partial(
    pl.kernel,
    out_type=jax.ShapeDtypeStruct((_F, _B), jnp.float32),
    mesh=plsc.VectorSubcoreMesh(core_axis_name="c", subcore_axis_name="s"),
    compiler_params=pltpu.CompilerParams(needs_layout_passes=False),
    scratch_types=[
        pltpu.VMEM((_T, _BPW), jnp.int32),
        pltpu.VMEM((_F, _V), jnp.float32),
        pltpu.VMEM((_F, _BPW), jnp.float32),
        pltpu.SemaphoreType.DMA,
        pltpu.SemaphoreType.DMA,
        pltpu.SemaphoreType.DMA,
        pltpu.SemaphoreType.DMA,
    ],
)(_sc_body)


def kernel(inputs, table):
    out_t = _sc_kernel(inputs.T.astype(jnp.int32), table.T)
    return out_t.T
